# Initial kernel scaffold; baseline (speedup 1.0000x reference)
import jax
import jax.numpy as jnp
from jax.experimental import pallas as pl


def kernel(conv1_w, conv1_b, conv2_w, conv2_b, tail_w, tail_b, mri_input, dna):
    raise NotImplementedError("write your pallas kernel here")


# speedup vs baseline: 3.3240x; 3.3240x over previous
import functools

import jax
import jax.numpy as jnp
from jax.experimental import pallas as pl
from jax.experimental.pallas import tpu as pltpu


def _offsets(flat):
    dna = [128, 64, 32, 16, 8, 6]
    shapes = [("fc1", flat, 128), ("fc2", 128, 6)]
    shapes += [(f"dna{i}", dna[i], dna[i + 1]) for i in range(5)]
    shapes += [("wv", 12, 12), ("wo", 12, 12), ("wg", 12, 2), ("whead", 6, 5)]
    off, table = 0, {}
    for name, K, N in shapes:
        table[name] = (off, K, N)
        off += -(-K // 8) * 8
    return table, off


def _fused_kernel(x_ref, dna_ref, wb1_ref, b1_ref, wb2_ref, b2_ref,
                  w1_ref, slab_ref, bias_ref, out_ref, *, G, H, W, woff):
    H2, W2 = H // 2, W // 2
    H4, W4 = H // 4, W // 4

    x = x_ref[...]
    acc = jnp.dot(x[:, 0:H, :].reshape(G * H, W * 3), wb1_ref[0],
                  preferred_element_type=jnp.float32)
    acc += jnp.dot(x[:, 1:H + 1, :].reshape(G * H, W * 3), wb1_ref[1],
                   preferred_element_type=jnp.float32)
    acc += jnp.dot(x[:, 2:H + 2, :].reshape(G * H, W * 3), wb1_ref[2],
                   preferred_element_type=jnp.float32)
    y = jnp.maximum(acc + b1_ref[...], 0.0)
    y = jnp.max(y.reshape(G * H2, 2, W * 16), axis=1)
    y = jnp.max(y.reshape(G * H2, W2, 2, 16), axis=2).reshape(G, H2, W2 * 16)

    x2 = jnp.pad(y, ((0, 0), (1, 1), (0, 0))).astype(jnp.bfloat16)
    acc = jnp.dot(x2[:, 0:H2, :].reshape(G * H2, W2 * 16), wb2_ref[0],
                  preferred_element_type=jnp.float32)
    acc += jnp.dot(x2[:, 1:H2 + 1, :].reshape(G * H2, W2 * 16), wb2_ref[1],
                   preferred_element_type=jnp.float32)
    acc += jnp.dot(x2[:, 2:H2 + 2, :].reshape(G * H2, W2 * 16), wb2_ref[2],
                   preferred_element_type=jnp.float32)
    y = jnp.maximum(acc + b2_ref[...], 0.0)
    y = jnp.max(y.reshape(G * H4, 2, W2 * 32), axis=1)
    y = jnp.max(y.reshape(G * H4, W4, 2, 32), axis=2).reshape(G, H4, W4 * 32)

    p = y.astype(jnp.bfloat16)
    h1 = jnp.dot(p[:, 0, :], w1_ref[0], preferred_element_type=jnp.float32)
    for hh in range(1, H4):
        h1 += jnp.dot(p[:, hh, :], w1_ref[hh], preferred_element_type=jnp.float32)
    h1 = jnp.maximum(h1 + bias_ref[0:1, :128], 0.0)

    def seg(name):
        off, K, N = woff[name]
        return slab_ref[off:off + K, :N]

    mri = (jnp.dot(h1, seg("fc2"), preferred_element_type=jnp.float32)
           + bias_ref[1:2, :6])

    d = dna_ref[...]
    for i in range(5):
        d = (jnp.dot(d, seg(f"dna{i}"), preferred_element_type=jnp.float32)
             + bias_ref[2 + i:3 + i, :woff[f"dna{i}"][2]])
        if i < 4:
            d = jnp.maximum(d, 0.0)

    offv, _, _ = woff["wv"]
    v = (jnp.dot(mri, slab_ref[offv:offv + 6, :12],
                 preferred_element_type=jnp.float32)
         + jnp.dot(d, slab_ref[offv + 6:offv + 12, :12],
                   preferred_element_type=jnp.float32)
         + bias_ref[7:8, :12])
    attn = (jnp.dot(v, seg("wo"), preferred_element_type=jnp.float32)
            + bias_ref[8:9, :12])

    g = (jnp.dot(attn, seg("wg"), preferred_element_type=jnp.float32)
         + bias_ref[9:10, :2])
    g = g - jnp.max(g, axis=1, keepdims=True)
    e = jnp.exp(g)
    gates = e / jnp.sum(e, axis=1, keepdims=True)
    comb = gates[:, 0:1] * mri + gates[:, 1:2] * d

    out_ref[...] = (jnp.dot(comb, seg("whead"), preferred_element_type=jnp.float32)
                    + bias_ref[10:11, :5])


def _band(w9, Wd, Cin, Cout):
    rows = []
    for dy in range(3):
        b = jnp.zeros((Wd * Cin, Wd * Cout), jnp.float32)
        for dx in range(3):
            b = b + jnp.kron(jnp.eye(Wd, Wd, k=1 - dx, dtype=jnp.float32),
                             w9[dy * 3 + dx])
        rows.append(b)
    return jnp.stack(rows).astype(jnp.bfloat16)


def kernel(conv1_w, conv1_b, conv2_w, conv2_b, tail_w, tail_b, mri_input, dna):
    B, Cin, H, W = mri_input.shape
    H4, W4 = H // 4, W // 4
    flat = H4 * W4 * 32
    G = 16 if B % 16 == 0 else B
    woff, _ = _offsets(flat)

    x = jnp.transpose(mri_input, (0, 2, 3, 1)).reshape(B, H, W * Cin)
    x = jnp.pad(x, ((0, 0), (1, 1), (0, 0))).astype(jnp.bfloat16)
    wb1 = _band(conv1_w, W, Cin, 16)
    wb2 = _band(conv2_w, W // 2, 16, 32)
    b1 = jnp.tile(conv1_b, (1, W))
    b2 = jnp.tile(conv2_b, (1, W // 2))
    w1 = tail_w[:flat, :128].reshape(H4, W4 * 32, 128).astype(jnp.bfloat16)
    fc2_off = woff["fc2"][0]
    slab = tail_w[fc2_off:fc2_off + (woff["whead"][0] - fc2_off) + 8, :]
    woff_rel = {k: (o - fc2_off, K, N) for k, (o, K, N) in woff.items()}

    fn = functools.partial(_fused_kernel, G=G, H=H, W=W, woff=woff_rel)
    out = pl.pallas_call(
        fn,
        out_shape=jax.ShapeDtypeStruct((B, 5), jnp.float32),
        grid=(B // G,),
        in_specs=[
            pl.BlockSpec((G, H + 2, W * Cin), lambda b: (b, 0, 0)),
            pl.BlockSpec((G, 128), lambda b: (b, 0)),
            pl.BlockSpec(wb1.shape, lambda b: (0, 0, 0)),
            pl.BlockSpec(b1.shape, lambda b: (0, 0)),
            pl.BlockSpec(wb2.shape, lambda b: (0, 0, 0)),
            pl.BlockSpec(b2.shape, lambda b: (0, 0)),
            pl.BlockSpec(w1.shape, lambda b: (0, 0, 0)),
            pl.BlockSpec(slab.shape, lambda b: (0, 0)),
            pl.BlockSpec(tail_b.shape, lambda b: (0, 0)),
        ],
        out_specs=pl.BlockSpec((G, 5), lambda b: (b, 0)),
        compiler_params=pltpu.CompilerParams(dimension_semantics=("parallel",)),
    )(x, dna, wb1, b1, wb2, b2, w1, slab, tail_b)
    return out[:, :3], out[:, 3:]
